# Initial kernel scaffold; baseline (speedup 1.0000x reference)
import jax
import jax.numpy as jnp
from jax.experimental import pallas as pl


def kernel(theta, edge_index, logc, u0):
    raise NotImplementedError("write your pallas kernel here")


# speedup vs baseline: 268.2066x; 268.2066x over previous
import functools

import jax
import jax.numpy as jnp
from jax import lax
from jax.experimental import pallas as pl
from jax.experimental.pallas import tpu as pltpu
from jax.experimental.pallas import tpu_sc as plsc

N = 100000
E = 6400000
LANES = 128
ROWS = E // LANES
TROWS = 784
NPAD = TROWS * LANES
NC, NS = 2, 16
NW = NC * NS
ZROWS = NPAD // NS
ROWS_LO = 1560
NHI = 10
FULL_CHUNKS = 97
TAIL_HI = 16
TAIL_LO = 8

_MASKHI = -65536


def _pack_body(th_ref, tab_ref):
    x = th_ref[...]
    s = jnp.sin(x)
    c = jnp.cos(x)
    su = lax.bitcast_convert_type(s.astype(jnp.bfloat16), jnp.uint16)
    cu = lax.bitcast_convert_type(c.astype(jnp.bfloat16), jnp.uint16)
    word = (su.astype(jnp.uint32) << 16) | cu.astype(jnp.uint32)
    tab_ref[...] = lax.bitcast_convert_type(word, jnp.int32)


_pack_call = pl.pallas_call(
    _pack_body,
    out_shape=jax.ShapeDtypeStruct((TROWS, LANES), jnp.int32),
)


def _combine_body(th_ref, m0, m1, d0, d1, lc_ref, u0_ref, w_ref, vc_ref, vs_ref):
    c = jnp.exp(lc_ref[0])
    u = u0_ref[0]
    sm = m0[...] + m1[...]
    dg = jnp.maximum(d0[...] + d1[...], 1.0)
    w_ref[...] = c * sm / dg
    x = th_ref[...]
    vc_ref[...] = u * jnp.cos(x)
    vs_ref[...] = u * jnp.sin(x)


_combine_call = pl.pallas_call(
    _combine_body,
    in_specs=[
        pl.BlockSpec(memory_space=pltpu.VMEM),
        pl.BlockSpec(memory_space=pltpu.VMEM),
        pl.BlockSpec(memory_space=pltpu.VMEM),
        pl.BlockSpec(memory_space=pltpu.VMEM),
        pl.BlockSpec(memory_space=pltpu.VMEM),
        pl.BlockSpec(memory_space=pltpu.SMEM),
        pl.BlockSpec(memory_space=pltpu.SMEM),
    ],
    out_shape=[
        jax.ShapeDtypeStruct((TROWS, LANES), jnp.float32),
        jax.ShapeDtypeStruct((TROWS, LANES), jnp.float32),
        jax.ShapeDtypeStruct((TROWS, LANES), jnp.float32),
    ],
)

_sc_mesh = plsc.VectorSubcoreMesh(core_axis_name="c", subcore_axis_name="s")


@functools.partial(
    pl.kernel,
    out_type=[
        jax.ShapeDtypeStruct((NC, NPAD), jnp.float32),
        jax.ShapeDtypeStruct((NC, NPAD), jnp.float32),
    ],
    mesh=_sc_mesh,
    compiler_params=pltpu.CompilerParams(needs_layout_passes=False),
    scratch_types=[
        pltpu.VMEM((NPAD,), jnp.int32),
        pltpu.VMEM((16 * LANES,), jnp.int32),
        pltpu.VMEM((16, LANES), jnp.int32),
        pltpu.VMEM((16 * LANES,), jnp.float32),
        pltpu.VMEM((LANES,), jnp.float32),
        pltpu.VMEM_SHARED((NPAD,), jnp.float32),
        pltpu.VMEM_SHARED((NPAD,), jnp.float32),
        pltpu.SemaphoreType.DMA,
    ],
)
def _sc_edges(tab_hbm, src_hbm, dst_hbm, zeros_hbm, outm_hbm, outd_hbm,
              tab, sidx, didx, mbuf, ones_row, accm, accd, sem):
    cid = lax.axis_index("c")
    sid = lax.axis_index("s")
    wid = cid * NS + sid
    ones = jnp.ones((16,), jnp.float32)

    pltpu.sync_copy(tab_hbm, tab)

    pltpu.sync_copy(zeros_hbm, accm.at[pl.ds(sid * ZROWS, ZROWS)])
    pltpu.sync_copy(zeros_hbm, accd.at[pl.ds(sid * ZROWS, ZROWS)])

    for k in range(LANES // 16):
        ones_row[pl.ds(k * 16, 16)] = ones

    plsc.subcore_barrier()

    def inner(i, carry):
        sv = sidx[pl.ds(i * 16, 16)]
        r = i >> 3
        cc = (i & 7) << 4
        dv = didx[r, pl.ds(cc, 16)]
        sw = plsc.load_gather(tab, [sv])
        dw = plsc.load_gather(tab, [dv])
        ssin = plsc.bitcast(sw & _MASKHI, jnp.float32)
        scos = plsc.bitcast(sw << 16, jnp.float32)
        dsin = plsc.bitcast(dw & _MASKHI, jnp.float32)
        dcos = plsc.bitcast(dw << 16, jnp.float32)
        m = ssin * dcos - scos * dsin
        mbuf[pl.ds(i * 16, 16)] = m
        return carry

    def fire(j, carry):
        pltpu.async_copy(mbuf.at[pl.ds(j * LANES, LANES)],
                         accm.at[didx.at[j]], sem, add=True)
        pltpu.async_copy(ones_row, accd.at[didx.at[j]], sem, add=True)
        return carry

    def drain(j, carry):
        pltpu.make_async_copy(mbuf.at[pl.ds(j * LANES, LANES)],
                              accm.at[didx.at[j]], sem).wait()
        pltpu.make_async_copy(ones_row, accd.at[didx.at[j]], sem).wait()
        return carry

    r0 = wid * ROWS_LO + 8 * jnp.minimum(wid, NHI)

    def chunk(g, carry):
        base = r0 + g * 16
        pltpu.sync_copy(src_hbm.at[pl.ds(base * LANES, 16 * LANES)], sidx)
        pltpu.sync_copy(dst_hbm.at[pl.ds(base, 16)], didx)
        lax.fori_loop(0, 16 * 8, inner, 0)
        lax.fori_loop(0, 16, fire, 0)
        lax.fori_loop(0, 16, drain, 0)
        return carry

    lax.fori_loop(0, FULL_CHUNKS, chunk, 0)

    tbase = r0 + FULL_CHUNKS * 16

    def tail(nr):
        pltpu.sync_copy(src_hbm.at[pl.ds(tbase * LANES, nr * LANES)],
                        sidx.at[pl.ds(0, nr * LANES)])
        pltpu.sync_copy(dst_hbm.at[pl.ds(tbase, nr)], didx.at[pl.ds(0, nr)])
        lax.fori_loop(0, nr * 8, inner, 0)
        lax.fori_loop(0, nr, fire, 0)
        lax.fori_loop(0, nr, drain, 0)

    @pl.when(wid < NHI)
    def _():
        tail(TAIL_HI)

    @pl.when(wid >= NHI)
    def _():
        tail(TAIL_LO)

    plsc.subcore_barrier()

    pltpu.sync_copy(accm.at[pl.ds(sid * ZROWS, ZROWS)],
                    outm_hbm.at[cid, pl.ds(sid * ZROWS, ZROWS)])
    pltpu.sync_copy(accd.at[pl.ds(sid * ZROWS, ZROWS)],
                    outd_hbm.at[cid, pl.ds(sid * ZROWS, ZROWS)])


def kernel(theta, edge_index, logc, u0):
    th = theta[:, 0]
    thp = jnp.pad(th, (0, NPAD - N)).reshape(TROWS, LANES)
    tab = _pack_call(thp).reshape(NPAD)
    src_f = edge_index[0]
    dst_2 = edge_index[1].reshape(ROWS, LANES)
    zeros = jnp.zeros((ZROWS,), jnp.float32)
    outm, outd = _sc_edges(tab, src_f, dst_2, zeros)
    m0 = outm[0].reshape(TROWS, LANES)
    m1 = outm[1].reshape(TROWS, LANES)
    d0 = outd[0].reshape(TROWS, LANES)
    d1 = outd[1].reshape(TROWS, LANES)
    lc = jnp.reshape(logc, (1,))
    uu = jnp.reshape(u0, (1,))
    wpad, vc, vs = _combine_call(thp, m0, m1, d0, d1, lc, uu)
    w = wpad.reshape(-1)[:N].reshape(N, 1)
    v = jnp.stack([vc.reshape(-1)[:N], vs.reshape(-1)[:N]], axis=-1)
    return w, v


# speedup vs baseline: 471.0510x; 1.7563x over previous
import functools

import jax
import jax.numpy as jnp
from jax import lax
from jax.experimental import pallas as pl
from jax.experimental.pallas import tpu as pltpu
from jax.experimental.pallas import tpu_sc as plsc

N = 100000
E = 6400000
LANES = 128
ROWS = E // LANES
TROWS = 784
NPAD = TROWS * LANES
NC, NS = 2, 16
NW = NC * NS
ZROWS = NPAD // NS
ROWS_LO = 1560
NHI = 10
FULL_CHUNKS = 97
TAIL_HI = 16
TAIL_LO = 8

_MASKHI = -65536


def _pack_body(th_ref, tab_ref):
    x = th_ref[...]
    s = jnp.sin(x)
    c = jnp.cos(x)
    su = lax.bitcast_convert_type(s.astype(jnp.bfloat16), jnp.uint16)
    cu = lax.bitcast_convert_type(c.astype(jnp.bfloat16), jnp.uint16)
    word = (su.astype(jnp.uint32) << 16) | cu.astype(jnp.uint32)
    tab_ref[...] = lax.bitcast_convert_type(word, jnp.int32)


_pack_call = pl.pallas_call(
    _pack_body,
    out_shape=jax.ShapeDtypeStruct((TROWS, LANES), jnp.int32),
)


def _combine_body(th_ref, m0, m1, d0, d1, lc_ref, u0_ref, w_ref, vc_ref, vs_ref):
    c = jnp.exp(lc_ref[0])
    u = u0_ref[0]
    sm = m0[...] + m1[...]
    dg = jnp.maximum(d0[...] + d1[...], 1.0)
    w_ref[...] = c * sm / dg
    x = th_ref[...]
    vc_ref[...] = u * jnp.cos(x)
    vs_ref[...] = u * jnp.sin(x)


_combine_call = pl.pallas_call(
    _combine_body,
    in_specs=[
        pl.BlockSpec(memory_space=pltpu.VMEM),
        pl.BlockSpec(memory_space=pltpu.VMEM),
        pl.BlockSpec(memory_space=pltpu.VMEM),
        pl.BlockSpec(memory_space=pltpu.VMEM),
        pl.BlockSpec(memory_space=pltpu.VMEM),
        pl.BlockSpec(memory_space=pltpu.SMEM),
        pl.BlockSpec(memory_space=pltpu.SMEM),
    ],
    out_shape=[
        jax.ShapeDtypeStruct((TROWS, LANES), jnp.float32),
        jax.ShapeDtypeStruct((TROWS, LANES), jnp.float32),
        jax.ShapeDtypeStruct((TROWS, LANES), jnp.float32),
    ],
)

_sc_mesh = plsc.VectorSubcoreMesh(core_axis_name="c", subcore_axis_name="s")


@functools.partial(
    pl.kernel,
    out_type=[
        jax.ShapeDtypeStruct((NC, NPAD), jnp.float32),
        jax.ShapeDtypeStruct((NC, NPAD), jnp.float32),
    ],
    mesh=_sc_mesh,
    compiler_params=pltpu.CompilerParams(needs_layout_passes=False),
    scratch_types=[
        pltpu.VMEM((NPAD,), jnp.int32),
        pltpu.VMEM((16 * LANES,), jnp.int32),
        pltpu.VMEM((16 * LANES,), jnp.int32),
        pltpu.VMEM((16, LANES), jnp.int32),
        pltpu.VMEM((16, LANES), jnp.int32),
        pltpu.VMEM((16, LANES), jnp.int32),
        pltpu.VMEM((16 * LANES,), jnp.float32),
        pltpu.VMEM((16 * LANES,), jnp.float32),
        pltpu.VMEM((16 * LANES,), jnp.float32),
        pltpu.VMEM((LANES,), jnp.float32),
        pltpu.VMEM_SHARED((NPAD,), jnp.float32),
        pltpu.VMEM_SHARED((NPAD,), jnp.float32),
        pltpu.SemaphoreType.DMA,
        pltpu.SemaphoreType.DMA,
        pltpu.SemaphoreType.DMA,
        pltpu.SemaphoreType.DMA,
        pltpu.SemaphoreType.DMA,
    ],
)
def _sc_edges(tab_hbm, src_hbm, dst_hbm, zeros_hbm, outm_hbm, outd_hbm,
              tab, sidx0, sidx1, didx0, didx1, didx2,
              mbuf0, mbuf1, mbuf2, ones_row, accm, accd,
              si0, si1, ss0, ss1, ss2):
    sidx_s = (sidx0, sidx1)
    didx_s = (didx0, didx1, didx2)
    mbuf_s = (mbuf0, mbuf1, mbuf2)
    si_s = (si0, si1)
    ss_s = (ss0, ss1, ss2)
    cid = lax.axis_index("c")
    sid = lax.axis_index("s")
    wid = cid * NS + sid
    ones = jnp.ones((16,), jnp.float32)

    pltpu.sync_copy(tab_hbm, tab)

    pltpu.sync_copy(zeros_hbm, accm.at[pl.ds(sid * ZROWS, ZROWS)])
    pltpu.sync_copy(zeros_hbm, accd.at[pl.ds(sid * ZROWS, ZROWS)])

    for k in range(LANES // 16):
        ones_row[pl.ds(k * 16, 16)] = ones

    plsc.subcore_barrier()

    r0 = wid * ROWS_LO + 8 * jnp.minimum(wid, NHI)

    def start_in(g, b2, b3):
        base = r0 + g * 16
        pltpu.async_copy(src_hbm.at[pl.ds(base * LANES, 16 * LANES)],
                         sidx_s[b2], si_s[b2])
        pltpu.async_copy(dst_hbm.at[pl.ds(base, 16)], didx_s[b3], si_s[b2])

    def wait_in(b2, b3):
        pltpu.make_async_copy(src_hbm.at[pl.ds(0, 16 * LANES)],
                              sidx_s[b2], si_s[b2]).wait()
        pltpu.make_async_copy(dst_hbm.at[pl.ds(0, 16)], didx_s[b3],
                              si_s[b2]).wait()

    def compute(b2, b3, nrows):
        sidx, didx, mbuf = sidx_s[b2], didx_s[b3], mbuf_s[b3]

        def inner(i, carry):
            sv = sidx[pl.ds(i * 16, 16)]
            r = i >> 3
            cc = (i & 7) << 4
            dv = didx[r, pl.ds(cc, 16)]
            sw = plsc.load_gather(tab, [sv])
            dw = plsc.load_gather(tab, [dv])
            ssin = plsc.bitcast(sw & _MASKHI, jnp.float32)
            scos = plsc.bitcast(sw << 16, jnp.float32)
            dsin = plsc.bitcast(dw & _MASKHI, jnp.float32)
            dcos = plsc.bitcast(dw << 16, jnp.float32)
            m = ssin * dcos - scos * dsin
            mbuf[pl.ds(i * 16, 16)] = m
            return carry

        lax.fori_loop(0, nrows * 8, inner, 0, unroll=4)

    def fire(b, nrows):
        didx, mbuf = didx_s[b], mbuf_s[b]

        def f(j, carry):
            pltpu.async_copy(mbuf.at[pl.ds(j * LANES, LANES)],
                             accm.at[didx.at[j]], ss_s[b], add=True)
            pltpu.async_copy(ones_row, accd.at[didx.at[j]],
                             ss_s[b], add=True)
            return carry

        lax.fori_loop(0, nrows, f, 0)

    def drain(b, nrows):
        didx, mbuf = didx_s[b], mbuf_s[b]

        def f(j, carry):
            pltpu.make_async_copy(mbuf.at[pl.ds(j * LANES, LANES)],
                                  accm.at[didx.at[j]], ss_s[b]).wait()
            pltpu.make_async_copy(ones_row, accd.at[didx.at[j]],
                                  ss_s[b]).wait()
            return carry

        lax.fori_loop(0, nrows, f, 0)

    start_in(0, 0, 0)

    def superstep(ss, carry):
        for b in range(6):
            g = ss * 6 + b

            @pl.when(jnp.logical_and(g >= 2, g < FULL_CHUNKS))
            def _(b=b):
                drain((b + 1) % 3, 16)

            @pl.when(g + 1 < FULL_CHUNKS)
            def _(b=b, g=g):
                start_in(g + 1, (b + 1) % 2, (b + 1) % 3)

            @pl.when(g < FULL_CHUNKS)
            def _(b=b):
                wait_in(b % 2, b % 3)
                compute(b % 2, b % 3, 16)
                fire(b % 3, 16)
        return carry

    lax.fori_loop(0, (FULL_CHUNKS + 5) // 6, superstep, 0)
    drain((FULL_CHUNKS - 2) % 3, 16)
    drain((FULL_CHUNKS - 1) % 3, 16)

    tbase = r0 + FULL_CHUNKS * 16

    def tail(nr):
        pltpu.sync_copy(src_hbm.at[pl.ds(tbase * LANES, nr * LANES)],
                        sidx0.at[pl.ds(0, nr * LANES)])
        pltpu.sync_copy(dst_hbm.at[pl.ds(tbase, nr)],
                        didx0.at[pl.ds(0, nr)])
        compute(0, 0, nr)
        fire(0, nr)
        drain(0, nr)

    @pl.when(wid < NHI)
    def _():
        tail(TAIL_HI)

    @pl.when(wid >= NHI)
    def _():
        tail(TAIL_LO)

    plsc.subcore_barrier()

    pltpu.sync_copy(accm.at[pl.ds(sid * ZROWS, ZROWS)],
                    outm_hbm.at[cid, pl.ds(sid * ZROWS, ZROWS)])
    pltpu.sync_copy(accd.at[pl.ds(sid * ZROWS, ZROWS)],
                    outd_hbm.at[cid, pl.ds(sid * ZROWS, ZROWS)])


def kernel(theta, edge_index, logc, u0):
    th = theta[:, 0]
    thp = jnp.pad(th, (0, NPAD - N)).reshape(TROWS, LANES)
    tab = _pack_call(thp).reshape(NPAD)
    src_f = edge_index[0]
    dst_2 = edge_index[1].reshape(ROWS, LANES)
    zeros = jnp.zeros((ZROWS,), jnp.float32)
    outm, outd = _sc_edges(tab, src_f, dst_2, zeros)
    m0 = outm[0].reshape(TROWS, LANES)
    m1 = outm[1].reshape(TROWS, LANES)
    d0 = outd[0].reshape(TROWS, LANES)
    d1 = outd[1].reshape(TROWS, LANES)
    lc = jnp.reshape(logc, (1,))
    uu = jnp.reshape(u0, (1,))
    wpad, vc, vs = _combine_call(thp, m0, m1, d0, d1, lc, uu)
    w = wpad.reshape(-1)[:N].reshape(N, 1)
    v = jnp.stack([vc.reshape(-1)[:N], vs.reshape(-1)[:N]], axis=-1)
    return w, v
